# baseline (speedup 1.0000x reference)
import jax
import jax.numpy as jnp
from jax.experimental import pallas as pl
from jax.experimental.pallas import tpu as pltpu

_NPOINTS = 512
_LANES = 128
_CH = 32


def _amax_lanes(v):
    for s in (1, 2, 4, 8, 16, 32, 64):
        v = jnp.maximum(v, pltpu.roll(v, s, axis=1))
    return v


def _amin_lanes(v):
    for s in (1, 2, 4, 8, 16, 32, 64):
        v = jnp.minimum(v, pltpu.roll(v, s, axis=1))
    return v


def _asum_lanes(v):
    for s in (1, 2, 4, 8, 16, 32, 64):
        v = v + pltpu.roll(v, s, axis=1)
    return v


def _fps_pallas(xyz, npoints):
    B, N, _ = xyz.shape
    rows = N // _LANES
    ch = min(_CH, rows)
    nchunks = rows // ch
    chunk_elems = ch * _LANES
    xyzt = jnp.transpose(xyz, (2, 0, 1)).reshape(3, B, rows, _LANES)
    far0 = jax.random.randint(jax.random.key(1), (B,), 0, N).astype(jnp.int32)

    def body(far0_ref, xyzt_ref, out_ref, dist_ref):
        dist_ref[...] = jnp.full((B, rows, _LANES), 1e10, jnp.float32)
        lane1 = jax.lax.broadcasted_iota(jnp.int32, (1, _LANES), 1)
        b_iota = jax.lax.broadcasted_iota(jnp.int32, (1, B), 1)
        pos_iota = (
            jax.lax.broadcasted_iota(jnp.int32, (ch, _LANES), 0) * _LANES
            + jax.lax.broadcasted_iota(jnp.int32, (ch, _LANES), 1)
        )

        def gather_vec(b, r, l):
            sel = lane1 == l
            xr = jnp.where(sel, xyzt_ref[0, b, pl.ds(r, 1), :], 0.0)
            yr = jnp.where(sel, xyzt_ref[1, b, pl.ds(r, 1), :], 0.0)
            zr = jnp.where(sel, xyzt_ref[2, b, pl.ds(r, 1), :], 0.0)
            return _asum_lanes(xr), _asum_lanes(yr), _asum_lanes(zr)

        def iter_body(i, carry):
            new_carry = []
            for b in range(B):
                cxv, cyv, czv = carry[b]
                rm = jnp.full((ch, _LANES), -1.0, jnp.float32)
                ri = jnp.zeros((ch, _LANES), jnp.int32)
                for k in range(nchunks):
                    sl = pl.ds(k * ch, ch)
                    x = xyzt_ref[0, b, sl, :]
                    y = xyzt_ref[1, b, sl, :]
                    z = xyzt_ref[2, b, sl, :]
                    dx = x - cxv
                    dy = y - cyv
                    dz = z - czv
                    d = dx * dx + dy * dy + dz * dz
                    nd = jnp.minimum(dist_ref[b, sl, :], d)
                    dist_ref[b, sl, :] = nd
                    gt = nd > rm
                    rm = jnp.where(gt, nd, rm)
                    ri = jnp.where(gt, k, ri)
                mvec = _amax_lanes(jnp.max(rm, axis=0, keepdims=True))
                cand = jnp.where(rm == mvec, ri * chunk_elems + pos_iota, N)
                fvec = _amin_lanes(jnp.min(cand, axis=0, keepdims=True))
                out_ref[pl.ds(i + 1, 1), pl.ds(b, 1)] = fvec[:, :1]
                f_sc = jnp.max(fvec)
                new_carry.append(gather_vec(b, f_sc // _LANES, f_sc % _LANES))
            return tuple(new_carry)

        rec = jnp.zeros((1, B), jnp.int32)
        carry0 = []
        for b in range(B):
            f0 = far0_ref[b]
            rec = jnp.where(b_iota == b, f0, rec)
            carry0.append(gather_vec(b, f0 // _LANES, f0 % _LANES))
        out_ref[pl.ds(0, 1), :] = rec

        jax.lax.fori_loop(0, npoints, iter_body, tuple(carry0))

    out = pl.pallas_call(
        body,
        grid=(),
        in_specs=[
            pl.BlockSpec(memory_space=pltpu.SMEM),
            pl.BlockSpec(memory_space=pltpu.VMEM),
        ],
        out_specs=pl.BlockSpec(memory_space=pltpu.VMEM),
        out_shape=jax.ShapeDtypeStruct((npoints + 1, B), jnp.int32),
        scratch_shapes=[pltpu.VMEM((B, rows, _LANES), jnp.float32)],
    )(far0, xyzt)
    return out[:npoints].T


def kernel(xyz):
    return _fps_pallas(xyz, _NPOINTS)


# speedup vs baseline: 1.1590x; 1.1590x over previous
import jax
import jax.numpy as jnp
from jax.experimental import pallas as pl
from jax.experimental.pallas import tpu as pltpu

_NPOINTS = 512
_LANES = 128
_CH = 32


def _amax_lanes(v):
    for s in (1, 2, 4, 8, 16, 32, 64):
        v = jnp.maximum(v, pltpu.roll(v, s, axis=1))
    return v


def _asum_lanes(v):
    for s in (1, 2, 4, 8, 16, 32, 64):
        v = v + pltpu.roll(v, s, axis=1)
    return v


def _fps_pallas(xyz, npoints):
    B, N, _ = xyz.shape
    rows = N // _LANES
    ch = min(_CH, rows)
    nchunks = rows // ch
    chunk_elems = ch * _LANES
    xyzt = jnp.transpose(xyz, (2, 0, 1)).reshape(3, B, rows, _LANES)
    xyz3 = jnp.transpose(xyz.reshape(B, rows, _LANES, 3), (0, 1, 3, 2))
    far0 = jax.random.randint(jax.random.key(1), (B,), 0, N).astype(jnp.int32)

    def body(far0_ref, xyzt_ref, xyz3_ref, out_ref, dist_ref):
        dist_ref[...] = jnp.full((B, rows, _LANES), 1e10, jnp.float32)
        lane3 = jax.lax.broadcasted_iota(jnp.int32, (3, _LANES), 1)
        b_iota = jax.lax.broadcasted_iota(jnp.int32, (1, B), 1)
        pos_iota = (
            jax.lax.broadcasted_iota(jnp.int32, (ch, _LANES), 0) * _LANES
            + jax.lax.broadcasted_iota(jnp.int32, (ch, _LANES), 1)
        )

        def gather_vec(b, f):
            g = xyz3_ref[b, pl.ds(f // _LANES, 1)].reshape(3, _LANES)
            return _asum_lanes(jnp.where(lane3 == f % _LANES, g, 0.0))

        def iter_body(i, carry):
            new_f = []
            new_c = []
            for b in range(B):
                cv = carry[b]
                cxb = jnp.broadcast_to(cv[0:1, :], (ch, _LANES))
                cyb = jnp.broadcast_to(cv[1:2, :], (ch, _LANES))
                czb = jnp.broadcast_to(cv[2:3, :], (ch, _LANES))
                rm = jnp.full((ch, _LANES), -1.0, jnp.float32)
                ri = jnp.zeros((ch, _LANES), jnp.int32)
                for k in range(nchunks):
                    sl = pl.ds(k * ch, ch)
                    x = xyzt_ref[0, b, sl, :]
                    y = xyzt_ref[1, b, sl, :]
                    z = xyzt_ref[2, b, sl, :]
                    dx = x - cxb
                    dy = y - cyb
                    dz = z - czb
                    d = dx * dx + dy * dy + dz * dz
                    nd = jnp.minimum(dist_ref[b, sl, :], d)
                    dist_ref[b, sl, :] = nd
                    gt = nd > rm
                    rm = jnp.where(gt, nd, rm)
                    ri = jnp.where(gt, k, ri)
                mvec = _amax_lanes(jnp.max(rm, axis=0, keepdims=True))
                f_sc = jnp.min(jnp.where(rm == mvec, ri * chunk_elems + pos_iota, N))
                new_f.append(f_sc)
                new_c.append(gather_vec(b, f_sc))
            rec = jnp.zeros((1, B), jnp.int32)
            for b in range(B):
                rec = jnp.where(b_iota == b, new_f[b], rec)
            out_ref[pl.ds(i + 1, 1), :] = rec
            return tuple(new_c)

        rec = jnp.zeros((1, B), jnp.int32)
        carry0 = []
        for b in range(B):
            f0 = far0_ref[b]
            rec = jnp.where(b_iota == b, f0, rec)
            carry0.append(gather_vec(b, f0))
        out_ref[pl.ds(0, 1), :] = rec

        jax.lax.fori_loop(0, npoints, iter_body, tuple(carry0))

    out = pl.pallas_call(
        body,
        grid=(),
        in_specs=[
            pl.BlockSpec(memory_space=pltpu.SMEM),
            pl.BlockSpec(memory_space=pltpu.VMEM),
            pl.BlockSpec(memory_space=pltpu.VMEM),
        ],
        out_specs=pl.BlockSpec(memory_space=pltpu.VMEM),
        out_shape=jax.ShapeDtypeStruct((npoints + 1, B), jnp.int32),
        scratch_shapes=[pltpu.VMEM((B, rows, _LANES), jnp.float32)],
    )(far0, xyzt, xyz3)
    return out[:npoints].T


def kernel(xyz):
    return _fps_pallas(xyz, _NPOINTS)
